# Initial kernel scaffold; baseline (speedup 1.0000x reference)
import jax
import jax.numpy as jnp
from jax.experimental import pallas as pl


def kernel(user_feat0_ids, user_feat1_ids, target_item0_ids, target_item1_ids, pos_hist_item0_ids, pos_hist_item1_ids, neg_hist_item0_ids, neg_hist_item1_ids, user0_table, user1_table, item0_table, item1_table):
    raise NotImplementedError("write your pallas kernel here")


# speedup vs baseline: 2.7887x; 2.7887x over previous
import functools

import jax
import jax.numpy as jnp
from jax import lax
from jax.experimental import pallas as pl
from jax.experimental.pallas import tpu as pltpu
from jax.experimental.pallas import tpu_sc as plsc

B = 4096
L = 50
D = 64
BL = B * L

NC = 2
NS = 16
NW = NC * NS

TGT_PER_W = B // NW
ROWS_PER_W = BL // NW
CHUNK = 640
NCHUNK = ROWS_PER_W // CHUNK


def _sc_body(u0i, u1i, t0i, t1i, p0i, p1i, n0i, n1i,
             u0t, u1t, i0t, i1t,
             out_ut, out_pos, out_neg,
             idx_t, rows_t, idx_h, rows_h, sem):
    wid = lax.axis_index("s") * NC + lax.axis_index("c")

    tb = wid * TGT_PER_W
    for j, (ids, tab) in enumerate(((u0i, u0t), (u1i, u1t),
                                    (t0i, i0t), (t1i, i1t))):
        pltpu.sync_copy(ids.at[pl.ds(tb, TGT_PER_W)], idx_t)
        pltpu.async_copy(tab.at[idx_t], rows_t, sem).wait()
        pltpu.sync_copy(rows_t, out_ut.at[pl.ds(tb, TGT_PER_W),
                                          pl.ds(j * D, D)])

    hb = wid * ROWS_PER_W
    for ids, tab, out, col in ((p0i, i0t, out_pos, 0),
                               (p1i, i1t, out_pos, D),
                               (n0i, i0t, out_neg, 0),
                               (n1i, i1t, out_neg, D)):
        def chunk_body(i, carry, ids=ids, tab=tab, out=out, col=col):
            off = hb + i * CHUNK
            pltpu.sync_copy(ids.at[pl.ds(off, CHUNK)], idx_h)
            pltpu.async_copy(tab.at[idx_h], rows_h, sem).wait()
            pltpu.sync_copy(rows_h, out.at[pl.ds(off, CHUNK),
                                           pl.ds(col, D)])
            return carry

        lax.fori_loop(0, NCHUNK, chunk_body, 0)


@functools.partial(jax.jit, donate_argnums=())
def _run(u0i, u1i, t0i, t1i, p0i, p1i, n0i, n1i, u0t, u1t, i0t, i1t):
    mesh = plsc.VectorSubcoreMesh(core_axis_name="c", subcore_axis_name="s")
    k = pl.kernel(
        _sc_body,
        out_type=[
            jax.ShapeDtypeStruct((B, 4 * D), jnp.float32),
            jax.ShapeDtypeStruct((BL, 2 * D), jnp.float32),
            jax.ShapeDtypeStruct((BL, 2 * D), jnp.float32),
        ],
        mesh=mesh,
        compiler_params=pltpu.CompilerParams(use_tc_tiling_on_sc=False),
        scratch_types=[
            pltpu.VMEM((TGT_PER_W,), jnp.int32),
            pltpu.VMEM((TGT_PER_W, D), jnp.float32),
            pltpu.VMEM((CHUNK,), jnp.int32),
            pltpu.VMEM((CHUNK, D), jnp.float32),
            pltpu.SemaphoreType.DMA,
        ],
    )
    return k(u0i, u1i, t0i, t1i, p0i, p1i, n0i, n1i, u0t, u1t, i0t, i1t)


def kernel(user_feat0_ids, user_feat1_ids, target_item0_ids, target_item1_ids,
           pos_hist_item0_ids, pos_hist_item1_ids,
           neg_hist_item0_ids, neg_hist_item1_ids,
           user0_table, user1_table, item0_table, item1_table):
    out_ut, out_pos, out_neg = _run(
        user_feat0_ids, user_feat1_ids, target_item0_ids, target_item1_ids,
        pos_hist_item0_ids.reshape(BL), pos_hist_item1_ids.reshape(BL),
        neg_hist_item0_ids.reshape(BL), neg_hist_item1_ids.reshape(BL),
        user0_table, user1_table, item0_table, item1_table)
    return (out_ut,
            out_pos.reshape(B, L, 2 * D),
            out_neg.reshape(B, L, 2 * D))


# speedup vs baseline: 2.8654x; 1.0275x over previous
import functools

import jax
import jax.numpy as jnp
from jax import lax
from jax.experimental import pallas as pl
from jax.experimental.pallas import tpu as pltpu
from jax.experimental.pallas import tpu_sc as plsc

B = 4096
L = 50
D = 64
BL = B * L

NC = 2
NS = 16
NW = NC * NS

TGT_PER_W = B // NW
ROWS_PER_W = BL // NW
CHUNK = 800
NCHUNK = ROWS_PER_W // CHUNK
NPAIR = NCHUNK // 2


def _sc_body(u0i, u1i, t0i, t1i, p0i, p1i, n0i, n1i,
             u0t, u1t, i0t, i1t,
             out_ut, out_pos, out_neg,
             idx_a, idx_b, rows_a, rows_b,
             is_a, is_b, gs_a, gs_b, ws_a, ws_b):
    idxb = (idx_a, idx_b)
    isem = (is_a, is_b)
    rows = (rows_a, rows_b)
    gsem = (gs_a, gs_b)
    wsem = (ws_a, ws_b)

    wid = lax.axis_index("s") * NC + lax.axis_index("c")

    def idx_start(ids, off, n, b):
        pltpu.async_copy(ids.at[pl.ds(off, n)], idxb[b].at[pl.ds(0, n)],
                         isem[b])

    def idx_wait(ids, n, b):
        pltpu.make_async_copy(ids.at[pl.ds(0, n)], idxb[b].at[pl.ds(0, n)],
                              isem[b]).wait()

    def gather_start(tab, n, b):
        pltpu.async_copy(tab.at[idxb[b].at[pl.ds(0, n)]],
                         rows[b].at[pl.ds(0, n)], gsem[b])

    def gather_wait(tab, n, b):
        pltpu.make_async_copy(tab.at[idxb[b].at[pl.ds(0, n)]],
                              rows[b].at[pl.ds(0, n)], gsem[b]).wait()

    def write_start(out, row, col, n, b):
        pltpu.async_copy(rows[b].at[pl.ds(0, n)],
                         out.at[pl.ds(row, n), pl.ds(col, D)], wsem[b])

    def write_wait(out, col, n, b):
        pltpu.make_async_copy(rows[b].at[pl.ds(0, n)],
                              out.at[pl.ds(0, n), pl.ds(col, D)],
                              wsem[b]).wait()

    T = TGT_PER_W
    tb = wid * T
    idx_start(u0i, tb, T, 0)
    idx_start(u1i, tb, T, 1)
    idx_wait(u0i, T, 0)
    gather_start(u0t, T, 0)
    idx_wait(u1i, T, 1)
    gather_start(u1t, T, 1)
    gather_wait(u0t, T, 0)
    write_start(out_ut, tb, 0, T, 0)
    idx_start(t0i, tb, T, 0)
    gather_wait(u1t, T, 1)
    write_start(out_ut, tb, D, T, 1)
    idx_start(t1i, tb, T, 1)
    idx_wait(t0i, T, 0)
    write_wait(out_ut, 0, T, 0)
    gather_start(i0t, T, 0)
    idx_wait(t1i, T, 1)
    write_wait(out_ut, D, T, 1)
    gather_start(i1t, T, 1)
    gather_wait(i0t, T, 0)
    write_start(out_ut, tb, 2 * D, T, 0)
    gather_wait(i1t, T, 1)
    write_start(out_ut, tb, 3 * D, T, 1)

    pend = [(out_ut, 2 * D, T), (out_ut, 3 * D, T)]

    hb = wid * ROWS_PER_W
    for ids, tab, out, col in ((p0i, i0t, out_pos, 0),
                               (p1i, i1t, out_pos, D),
                               (n0i, i0t, out_neg, 0),
                               (n1i, i1t, out_neg, D)):
        idx_start(ids, hb, CHUNK, 0)
        idx_start(ids, hb + CHUNK, CHUNK, 1)

        idx_wait(ids, CHUNK, 0)
        write_wait(*pend[0], 0)
        gather_start(tab, CHUNK, 0)
        idx_wait(ids, CHUNK, 1)
        write_wait(*pend[1], 1)
        gather_start(tab, CHUNK, 1)
        gather_wait(tab, CHUNK, 0)
        write_start(out, hb, col, CHUNK, 0)
        idx_start(ids, hb + 2 * CHUNK, CHUNK, 0)
        gather_wait(tab, CHUNK, 1)
        write_start(out, hb + CHUNK, col, CHUNK, 1)
        idx_start(ids, hb + 3 * CHUNK, CHUNK, 1)

        def pair(j, carry, ids=ids, tab=tab, out=out, col=col):
            a_off = hb + (2 * j) * CHUNK
            b_off = a_off + CHUNK
            idx_wait(ids, CHUNK, 0)
            write_wait(out, col, CHUNK, 0)
            gather_start(tab, CHUNK, 0)
            idx_wait(ids, CHUNK, 1)
            write_wait(out, col, CHUNK, 1)
            gather_start(tab, CHUNK, 1)
            pa = jnp.minimum(a_off + 2 * CHUNK, BL - CHUNK)
            pb = jnp.minimum(b_off + 2 * CHUNK, BL - CHUNK)
            gather_wait(tab, CHUNK, 0)
            write_start(out, a_off, col, CHUNK, 0)
            idx_start(ids, pa, CHUNK, 0)
            gather_wait(tab, CHUNK, 1)
            write_start(out, b_off, col, CHUNK, 1)
            idx_start(ids, pb, CHUNK, 1)
            return carry

        lax.fori_loop(1, NPAIR, pair, 0)

        idx_wait(ids, CHUNK, 0)
        idx_wait(ids, CHUNK, 1)
        pend = [(out, col, CHUNK), (out, col, CHUNK)]

    write_wait(*pend[0], 0)
    write_wait(*pend[1], 1)


@functools.partial(jax.jit, donate_argnums=())
def _run(u0i, u1i, t0i, t1i, p0i, p1i, n0i, n1i, u0t, u1t, i0t, i1t):
    mesh = plsc.VectorSubcoreMesh(core_axis_name="c", subcore_axis_name="s")
    k = pl.kernel(
        _sc_body,
        out_type=[
            jax.ShapeDtypeStruct((B, 4 * D), jnp.float32),
            jax.ShapeDtypeStruct((BL, 2 * D), jnp.float32),
            jax.ShapeDtypeStruct((BL, 2 * D), jnp.float32),
        ],
        mesh=mesh,
        compiler_params=pltpu.CompilerParams(use_tc_tiling_on_sc=False),
        scratch_types=[
            pltpu.VMEM((CHUNK,), jnp.int32),
            pltpu.VMEM((CHUNK,), jnp.int32),
            pltpu.VMEM((CHUNK, D), jnp.float32),
            pltpu.VMEM((CHUNK, D), jnp.float32),
            pltpu.SemaphoreType.DMA,
            pltpu.SemaphoreType.DMA,
            pltpu.SemaphoreType.DMA,
            pltpu.SemaphoreType.DMA,
            pltpu.SemaphoreType.DMA,
            pltpu.SemaphoreType.DMA,
        ],
    )
    return k(u0i, u1i, t0i, t1i, p0i, p1i, n0i, n1i, u0t, u1t, i0t, i1t)


def kernel(user_feat0_ids, user_feat1_ids, target_item0_ids, target_item1_ids,
           pos_hist_item0_ids, pos_hist_item1_ids,
           neg_hist_item0_ids, neg_hist_item1_ids,
           user0_table, user1_table, item0_table, item1_table):
    out_ut, out_pos, out_neg = _run(
        user_feat0_ids, user_feat1_ids, target_item0_ids, target_item1_ids,
        pos_hist_item0_ids.reshape(BL), pos_hist_item1_ids.reshape(BL),
        neg_hist_item0_ids.reshape(BL), neg_hist_item1_ids.reshape(BL),
        user0_table, user1_table, item0_table, item1_table)
    return (out_ut,
            out_pos.reshape(B, L, 2 * D),
            out_neg.reshape(B, L, 2 * D))


# speedup vs baseline: 4.0245x; 1.4045x over previous
import functools

import jax
import jax.numpy as jnp
from jax import lax
from jax.experimental import pallas as pl
from jax.experimental.pallas import tpu as pltpu
from jax.experimental.pallas import tpu_sc as plsc

B = 4096
L = 50
D = 64
BL = B * L

NC = 2
NS = 16
NW = NC * NS

TGT_PER_W = B // NW
ROWS_PER_W = BL // NW
CHUNK = 800
NCHUNK = ROWS_PER_W // CHUNK
NPAIR = NCHUNK // 2


def _sc_body(u0i, u1i, t0i, t1i, p0i, p1i, n0i, n1i,
             u0t, u1t, i0t, i1t,
             out_ut, out_pos, out_neg,
             idx_a, idx_b, rows_a, rows_b,
             is_a, is_b, gs_a, gs_b, ws_a, ws_b):
    idxb = (idx_a, idx_b)
    isem = (is_a, is_b)
    rows = (rows_a, rows_b)
    gsem = (gs_a, gs_b)
    wsem = (ws_a, ws_b)

    wid = lax.axis_index("s") * NC + lax.axis_index("c")

    def idx_start(ids, off, n, b):
        pltpu.async_copy(ids.at[pl.ds(off, n)], idxb[b].at[pl.ds(0, n)],
                         isem[b])

    def idx_wait(ids, n, b):
        pltpu.make_async_copy(ids.at[pl.ds(0, n)], idxb[b].at[pl.ds(0, n)],
                              isem[b]).wait()

    def gather_start(tab, n, b):
        pltpu.async_copy(tab.at[idxb[b].at[pl.ds(0, n)]],
                         rows[b].at[pl.ds(0, n)], gsem[b])

    def gather_wait(tab, n, b):
        pltpu.make_async_copy(tab.at[idxb[b].at[pl.ds(0, n)]],
                              rows[b].at[pl.ds(0, n)], gsem[b]).wait()

    def write_start(out, row, col, n, b):
        pltpu.async_copy(rows[b].at[pl.ds(0, n)],
                         out.at[pl.ds(row, n), pl.ds(col, D)], wsem[b])

    def write_wait(out, col, n, b):
        pltpu.make_async_copy(rows[b].at[pl.ds(0, n)],
                              out.at[pl.ds(0, n), pl.ds(col, D)],
                              wsem[b]).wait()

    T = TGT_PER_W
    tb = wid * T
    idx_start(u0i, tb, T, 0)
    idx_start(u1i, tb, T, 1)
    idx_wait(u0i, T, 0)
    gather_start(u0t, T, 0)
    idx_wait(u1i, T, 1)
    gather_start(u1t, T, 1)
    gather_wait(u0t, T, 0)
    write_start(out_ut, tb, 0, T, 0)
    idx_start(t0i, tb, T, 0)
    gather_wait(u1t, T, 1)
    write_start(out_ut, tb, D, T, 1)
    idx_start(t1i, tb, T, 1)
    idx_wait(t0i, T, 0)
    write_wait(out_ut, 0, T, 0)
    gather_start(i0t, T, 0)
    idx_wait(t1i, T, 1)
    write_wait(out_ut, D, T, 1)
    gather_start(i1t, T, 1)
    gather_wait(i0t, T, 0)
    write_start(out_ut, tb, 2 * D, T, 0)
    gather_wait(i1t, T, 1)
    write_start(out_ut, tb, 3 * D, T, 1)

    pend = [(out_ut, 2 * D, T), (out_ut, 3 * D, T)]

    hb = wid * ROWS_PER_W
    for ids, tab, out, col in ((p0i, i0t, out_pos, 0),
                               (p1i, i1t, out_pos, D),
                               (n0i, i0t, out_neg, 0),
                               (n1i, i1t, out_neg, D)):
        idx_start(ids, hb, CHUNK, 0)
        idx_start(ids, hb + CHUNK, CHUNK, 1)

        idx_wait(ids, CHUNK, 0)
        write_wait(*pend[0], 0)
        gather_start(tab, CHUNK, 0)
        idx_wait(ids, CHUNK, 1)
        write_wait(*pend[1], 1)
        gather_start(tab, CHUNK, 1)
        gather_wait(tab, CHUNK, 0)
        write_start(out, hb, col, CHUNK, 0)
        idx_start(ids, hb + 2 * CHUNK, CHUNK, 0)
        gather_wait(tab, CHUNK, 1)
        write_start(out, hb + CHUNK, col, CHUNK, 1)
        idx_start(ids, hb + 3 * CHUNK, CHUNK, 1)

        def pair(j, carry, ids=ids, tab=tab, out=out, col=col):
            a_off = hb + (2 * j) * CHUNK
            b_off = a_off + CHUNK
            idx_wait(ids, CHUNK, 0)
            write_wait(out, col, CHUNK, 0)
            gather_start(tab, CHUNK, 0)
            idx_wait(ids, CHUNK, 1)
            write_wait(out, col, CHUNK, 1)
            gather_start(tab, CHUNK, 1)
            pa = jnp.minimum(a_off + 2 * CHUNK, BL - CHUNK)
            pb = jnp.minimum(b_off + 2 * CHUNK, BL - CHUNK)
            gather_wait(tab, CHUNK, 0)
            write_start(out, a_off, col, CHUNK, 0)
            idx_start(ids, pa, CHUNK, 0)
            gather_wait(tab, CHUNK, 1)
            write_start(out, b_off, col, CHUNK, 1)
            idx_start(ids, pb, CHUNK, 1)
            return carry

        lax.fori_loop(1, NPAIR, pair, 0)

        idx_wait(ids, CHUNK, 0)
        idx_wait(ids, CHUNK, 1)
        pend = [(out, col, CHUNK), (out, col, CHUNK)]

    write_wait(*pend[0], 0)
    write_wait(*pend[1], 1)


@functools.partial(jax.jit, donate_argnums=())
def _run(u0i, u1i, t0i, t1i, p0i, p1i, n0i, n1i, u0t, u1t, i0t, i1t):
    mesh = plsc.VectorSubcoreMesh(core_axis_name="c", subcore_axis_name="s")
    k = pl.kernel(
        _sc_body,
        out_type=[
            jax.ShapeDtypeStruct((B, 4 * D), jnp.float32),
            jax.ShapeDtypeStruct((BL, 2 * D), jnp.float32),
            jax.ShapeDtypeStruct((BL, 2 * D), jnp.float32),
        ],
        mesh=mesh,
        compiler_params=pltpu.CompilerParams(use_tc_tiling_on_sc=False),
        scratch_types=[
            pltpu.VMEM((CHUNK,), jnp.int32),
            pltpu.VMEM((CHUNK,), jnp.int32),
            pltpu.VMEM((CHUNK, D), jnp.float32),
            pltpu.VMEM((CHUNK, D), jnp.float32),
            pltpu.SemaphoreType.DMA,
            pltpu.SemaphoreType.DMA,
            pltpu.SemaphoreType.DMA,
            pltpu.SemaphoreType.DMA,
            pltpu.SemaphoreType.DMA,
            pltpu.SemaphoreType.DMA,
        ],
    )
    return k(u0i, u1i, t0i, t1i, p0i, p1i, n0i, n1i, u0t, u1t, i0t, i1t)


def kernel(user_feat0_ids, user_feat1_ids, target_item0_ids, target_item1_ids,
           pos_hist_item0_ids, pos_hist_item1_ids,
           neg_hist_item0_ids, neg_hist_item1_ids,
           user0_table, user1_table, item0_table, item1_table):
    out_ut, out_pos, out_neg = _run(
        user_feat0_ids, user_feat1_ids, target_item0_ids, target_item1_ids,
        pos_hist_item0_ids.T.reshape(BL), pos_hist_item1_ids.T.reshape(BL),
        neg_hist_item0_ids.T.reshape(BL), neg_hist_item1_ids.T.reshape(BL),
        user0_table, user1_table, item0_table, item1_table)
    return (out_ut,
            out_pos.reshape(L, B, 2 * D).transpose(1, 0, 2),
            out_neg.reshape(L, B, 2 * D).transpose(1, 0, 2))


# speedup vs baseline: 4.3011x; 1.0687x over previous
import functools

import jax
import jax.numpy as jnp
from jax import lax
from jax.experimental import pallas as pl
from jax.experimental.pallas import tpu as pltpu
from jax.experimental.pallas import tpu_sc as plsc

B = 4096
L = 50
D = 64
BL = B * L

NC = 2
NS = 16
NW = NC * NS

TGT_PER_W = B // NW
ROWS_PER_W = BL // NW
CHUNK = 800
NCHUNK = ROWS_PER_W // CHUNK
NPAIR = NCHUNK // 2


def _sc_body(u0i, u1i, t0i, t1i, p0i, p1i, n0i, n1i,
             u0t, u1t, i0t, i1t,
             out_ut, out_pos, out_neg,
             idx_a, idx_b, rows_a, rows_b,
             is_a, is_b, gs_a, gs_b, ws_a, ws_b):
    idxb = (idx_a, idx_b)
    isem = (is_a, is_b)
    rows = (rows_a, rows_b)
    gsem = (gs_a, gs_b)
    wsem = (ws_a, ws_b)

    wid = lax.axis_index("s") * NC + lax.axis_index("c")

    def idx_start(ids, off, n, b):
        pltpu.async_copy(ids.at[pl.ds(off, n)], idxb[b].at[pl.ds(0, n)],
                         isem[b])

    def idx_wait(ids, n, b):
        pltpu.make_async_copy(ids.at[pl.ds(0, n)], idxb[b].at[pl.ds(0, n)],
                              isem[b]).wait()

    def gather_start(tab, n, b):
        pltpu.async_copy(tab.at[idxb[b].at[pl.ds(0, n)]],
                         rows[b].at[pl.ds(0, n)], gsem[b])

    def gather_wait(tab, n, b):
        pltpu.make_async_copy(tab.at[idxb[b].at[pl.ds(0, n)]],
                              rows[b].at[pl.ds(0, n)], gsem[b]).wait()

    def write_start(out, row, col, n, b):
        pltpu.async_copy(rows[b].at[pl.ds(0, n)],
                         out.at[pl.ds(row, n), pl.ds(col, D)], wsem[b])

    def write_wait(out, col, n, b):
        pltpu.make_async_copy(rows[b].at[pl.ds(0, n)],
                              out.at[pl.ds(0, n), pl.ds(col, D)],
                              wsem[b]).wait()

    T = TGT_PER_W
    tb = wid * T
    idx_start(u0i, tb, T, 0)
    idx_start(u1i, tb, T, 1)
    idx_wait(u0i, T, 0)
    gather_start(u0t, T, 0)
    idx_wait(u1i, T, 1)
    gather_start(u1t, T, 1)
    gather_wait(u0t, T, 0)
    write_start(out_ut, tb, 0, T, 0)
    idx_start(t0i, tb, T, 0)
    gather_wait(u1t, T, 1)
    write_start(out_ut, tb, D, T, 1)
    idx_start(t1i, tb, T, 1)
    idx_wait(t0i, T, 0)
    write_wait(out_ut, 0, T, 0)
    gather_start(i0t, T, 0)
    idx_wait(t1i, T, 1)
    write_wait(out_ut, D, T, 1)
    gather_start(i1t, T, 1)
    gather_wait(i0t, T, 0)
    write_start(out_ut, tb, 2 * D, T, 0)
    gather_wait(i1t, T, 1)
    write_start(out_ut, tb, 3 * D, T, 1)

    pend = [(out_ut, 2 * D, T), (out_ut, 3 * D, T)]

    hb = wid * ROWS_PER_W
    for ids, tab, out, col in ((p0i, i0t, out_pos, 0),
                               (p1i, i1t, out_pos, D),
                               (n0i, i0t, out_neg, 0),
                               (n1i, i1t, out_neg, D)):
        idx_start(ids, hb, CHUNK, 0)
        idx_start(ids, hb + CHUNK, CHUNK, 1)

        idx_wait(ids, CHUNK, 0)
        write_wait(*pend[0], 0)
        gather_start(tab, CHUNK, 0)
        idx_wait(ids, CHUNK, 1)
        write_wait(*pend[1], 1)
        gather_start(tab, CHUNK, 1)
        gather_wait(tab, CHUNK, 0)
        write_start(out, hb, col, CHUNK, 0)
        idx_start(ids, hb + 2 * CHUNK, CHUNK, 0)
        gather_wait(tab, CHUNK, 1)
        write_start(out, hb + CHUNK, col, CHUNK, 1)
        idx_start(ids, hb + 3 * CHUNK, CHUNK, 1)

        def pair(j, carry, ids=ids, tab=tab, out=out, col=col):
            a_off = hb + (2 * j) * CHUNK
            b_off = a_off + CHUNK
            idx_wait(ids, CHUNK, 0)
            write_wait(out, col, CHUNK, 0)
            gather_start(tab, CHUNK, 0)
            idx_wait(ids, CHUNK, 1)
            write_wait(out, col, CHUNK, 1)
            gather_start(tab, CHUNK, 1)
            pa = jnp.minimum(a_off + 2 * CHUNK, BL - CHUNK)
            pb = jnp.minimum(b_off + 2 * CHUNK, BL - CHUNK)
            gather_wait(tab, CHUNK, 0)
            write_start(out, a_off, col, CHUNK, 0)
            idx_start(ids, pa, CHUNK, 0)
            gather_wait(tab, CHUNK, 1)
            write_start(out, b_off, col, CHUNK, 1)
            idx_start(ids, pb, CHUNK, 1)
            return carry

        lax.fori_loop(1, NPAIR, pair, 0)

        idx_wait(ids, CHUNK, 0)
        idx_wait(ids, CHUNK, 1)
        pend = [(out, col, CHUNK), (out, col, CHUNK)]

    write_wait(*pend[0], 0)
    write_wait(*pend[1], 1)


@functools.partial(jax.jit, donate_argnums=())
def _run(u0i, u1i, t0i, t1i, p0i, p1i, n0i, n1i, u0t, u1t, i0t, i1t):
    mesh = plsc.VectorSubcoreMesh(core_axis_name="c", subcore_axis_name="s")
    k = pl.kernel(
        _sc_body,
        out_type=[
            jax.ShapeDtypeStruct((B, 4 * D), jnp.float32),
            jax.ShapeDtypeStruct((BL, 2 * D), jnp.float32),
            jax.ShapeDtypeStruct((BL, 2 * D), jnp.float32),
        ],
        mesh=mesh,
        compiler_params=pltpu.CompilerParams(use_tc_tiling_on_sc=False),
        scratch_types=[
            pltpu.VMEM((CHUNK,), jnp.int32),
            pltpu.VMEM((CHUNK,), jnp.int32),
            pltpu.VMEM((CHUNK, D), jnp.float32),
            pltpu.VMEM((CHUNK, D), jnp.float32),
            pltpu.SemaphoreType.DMA,
            pltpu.SemaphoreType.DMA,
            pltpu.SemaphoreType.DMA,
            pltpu.SemaphoreType.DMA,
            pltpu.SemaphoreType.DMA,
            pltpu.SemaphoreType.DMA,
        ],
    )
    return k(u0i, u1i, t0i, t1i, p0i, p1i, n0i, n1i, u0t, u1t, i0t, i1t)


def kernel(user_feat0_ids, user_feat1_ids, target_item0_ids, target_item1_ids,
           pos_hist_item0_ids, pos_hist_item1_ids,
           neg_hist_item0_ids, neg_hist_item1_ids,
           user0_table, user1_table, item0_table, item1_table):
    def _tab(t):
        vp = t.shape[0] + (-t.shape[0]) % 8
        return jnp.pad(t, ((0, vp - t.shape[0]), (0, D))).reshape(2 * vp, D)

    out_ut, out_pos, out_neg = _run(
        user_feat0_ids * 2, user_feat1_ids * 2,
        target_item0_ids * 2, target_item1_ids * 2,
        pos_hist_item0_ids.T.reshape(BL) * 2,
        pos_hist_item1_ids.T.reshape(BL) * 2,
        neg_hist_item0_ids.T.reshape(BL) * 2,
        neg_hist_item1_ids.T.reshape(BL) * 2,
        _tab(user0_table), _tab(user1_table),
        _tab(item0_table), _tab(item1_table))
    return (out_ut,
            out_pos.reshape(L, B, 2 * D).transpose(1, 0, 2),
            out_neg.reshape(L, B, 2 * D).transpose(1, 0, 2))
